# Initial kernel scaffold; baseline (speedup 1.0000x reference)
import jax
import jax.numpy as jnp
from jax.experimental import pallas as pl


def kernel(y_pred):
    raise NotImplementedError("write your pallas kernel here")


# speedup vs baseline: 1.9503x; 1.9503x over previous
import functools

import jax
import jax.numpy as jnp
from jax import lax
from jax.experimental import pallas as pl
from jax.experimental.pallas import tpu as pltpu
from jax.experimental.pallas import tpu_sc as plsc

_CONF = 0.01
_IOU_T = 0.45
_TOPK = 200
_NMSMAX = 400
_K = 800
_NC = 20
_NI = 8
_NA = 20000
_ROWS = 160
_NAP = _ROWS * 128
_PAD_IDX = _NAP - 1
_NEG = -1e30


def _rm_excl_cumsum(x):
    r = x
    for k in (1, 2, 4, 8, 16, 32, 64):
        r = r + jnp.concatenate(
            [jnp.zeros((_ROWS, k), jnp.float32), r[:, : 128 - k]], axis=1)
    row_tot = jnp.sum(x, axis=1, keepdims=True)
    t = row_tot
    k = 1
    while k < _ROWS:
        t = t + jnp.concatenate(
            [jnp.zeros((k, 1), jnp.float32), t[: _ROWS - k, :]], axis=0)
        k *= 2
    return (r - x) + (t - row_tot)


def _select_kernel(ypt_ref, pm_ref, sth_ref, bxs_ref):
    c = pl.program_id(1)

    @pl.when(c == 0)
    def _decode():
        cx_p = ypt_ref[0, 21]
        cy_p = ypt_ref[0, 22]
        w_p = ypt_ref[0, 23]
        h_p = ypt_ref[0, 24]
        x1a = ypt_ref[0, 25]
        y1a = ypt_ref[0, 26]
        x2a = ypt_ref[0, 27]
        y2a = ypt_ref[0, 28]
        cxv = ypt_ref[0, 29]
        cyv = ypt_ref[0, 30]
        wv = ypt_ref[0, 31]
        hv = ypt_ref[0, 32]
        wa = x2a - x1a
        ha = y2a - y1a
        cxa = (x2a + x1a) * 0.5
        cya = (y2a + y1a) * 0.5
        cx = cx_p * cxv * wa + cxa
        cy = cy_p * cyv * ha + cya
        w = jnp.exp(w_p * wv) * wa
        h = jnp.exp(h_p * hv) * ha
        bxs_ref[0, 0] = (cx - 0.5 * w) * 512.0
        bxs_ref[0, 1] = (cy - 0.5 * h) * 512.0
        bxs_ref[0, 2] = (cx + 0.5 * w) * 512.0
        bxs_ref[0, 3] = (cy + 0.5 * h) * 512.0

    v = ypt_ref[0, pl.ds(1 + c, 1)]
    v = jnp.reshape(v, (_ROWS, 128))
    s = jnp.where(v > _CONF, v, -jnp.inf)
    sth_ref[0, 0] = s

    sbits = lax.bitcast_convert_type(s, jnp.int32)
    n_fin = jnp.sum((v > _CONF).astype(jnp.int32))
    target = jnp.minimum(_K, n_fin)

    def bs_body(_, carry):
        lo, hi = carry
        m = lo + (hi - lo) // 2
        cnt = jnp.sum((sbits > m).astype(jnp.int32))
        big = cnt >= target
        return jnp.where(big, m, lo), jnp.where(big, hi, m)

    lo0 = jnp.int32(0)
    hi0 = jnp.int32(0x7F800000)
    _, t_bits = lax.fori_loop(0, 31, bs_body, (lo0, hi0))

    gt = sbits > t_bits
    eqm = sbits == t_bits
    n_gt = jnp.sum(gt.astype(jnp.int32))
    k_eq = (target - n_gt).astype(jnp.float32)
    eq_rank = _rm_excl_cumsum(eqm.astype(jnp.float32))
    sel = gt | (eqm & (eq_rank < k_eq))
    p = _rm_excl_cumsum(sel.astype(jnp.float32)).astype(jnp.int32)
    pm_ref[0, 0] = jnp.where(sel, p, _NAP)


def _sc_compact_call(pm, sth, bxs):
    mesh = plsc.VectorSubcoreMesh(core_axis_name="c", subcore_axis_name="s")

    @functools.partial(
        pl.kernel,
        mesh=mesh,
        out_type=jax.ShapeDtypeStruct((_NI * _NC * 5 * _K,), jnp.float32),
        scratch_types=[
            pltpu.VMEM((_NAP,), jnp.int32),
            pltpu.VMEM((_K,), jnp.int32),
            pltpu.VMEM((_NAP,), jnp.float32),
            pltpu.VMEM((5 * _K,), jnp.float32),
        ],
        compiler_params=pltpu.CompilerParams(needs_layout_passes=False),
    )
    def k(pm_hbm, sth_hbm, bxs_hbm, out_hbm, pm_v, cidx_v, plane_v, obuf_v):
        wid = lax.axis_index("s") * 2 + lax.axis_index("c")

        def unit_body(u, carry):
            unit = u * 32 + wid
            img = unit // _NC
            pltpu.sync_copy(pm_hbm.at[pl.ds(unit * _NAP, _NAP)], pm_v)

            def initb(j, carry2):
                cidx_v[pl.ds(j * 16, 16)] = jnp.full((16,), _PAD_IDX, jnp.int32)
                return carry2

            lax.fori_loop(0, _K // 16, initb, 0)

            def scat(i, carry2):
                pv = pm_v[pl.ds(i * 16, 16)]
                mask = pv < _K
                gi = lax.iota(jnp.int32, 16) + i * 16
                plsc.store_scatter(cidx_v, [pv], gi, mask=mask)
                return carry2

            lax.fori_loop(0, _NAP // 16, scat, 0)

            for p in range(5):
                if p == 0:
                    pltpu.sync_copy(
                        sth_hbm.at[pl.ds(unit * _NAP, _NAP)], plane_v)
                else:
                    pltpu.sync_copy(
                        bxs_hbm.at[pl.ds((img * 4 + p - 1) * _NAP, _NAP)],
                        plane_v)

                def gath(j, carry2):
                    iv = cidx_v[pl.ds(j * 16, 16)]
                    obuf_v[pl.ds(p * _K + j * 16, 16)] = plsc.load_gather(
                        plane_v, [iv])
                    return carry2

                lax.fori_loop(0, _K // 16, gath, 0)

            pltpu.sync_copy(obuf_v, out_hbm.at[pl.ds(unit * 5 * _K, 5 * _K)])
            return carry

        lax.fori_loop(0, 5, unit_body, 0)

    return k(pm, sth, bxs)


def _nms_kernel(comp_ref, out_ref, sup_ref, msc_ref, sb_ref):
    row_i = lax.broadcasted_iota(jnp.int32, (_K, _K), 0).astype(jnp.float32)
    col_i = lax.broadcasted_iota(jnp.int32, (_K, _K), 1).astype(jnp.float32)
    lane_i = lax.broadcasted_iota(jnp.int32, (1, _K), 1).astype(jnp.float32)

    def class_body(c, carry):
        cp = comp_ref[0, pl.ds(c, 1)]
        cp = jnp.reshape(cp, (5, _K))
        score = cp[0:1, :]
        finite = score > _NEG
        n_sel = jnp.sum(finite.astype(jnp.int32)).astype(jnp.float32)
        cp = jnp.where(finite, cp, 0.0)
        s_row = cp[0:1, :]
        cpt = lax.transpose(cp, (1, 0))
        s_col = cpt[:, 0:1]
        m2 = (s_row > s_col) | ((s_row == s_col) & (col_i < row_i))
        rank_col = jnp.sum(m2.astype(jnp.float32), axis=1, keepdims=True)
        p2t = (rank_col == col_i).astype(jnp.float32)
        sorted5 = jax.lax.dot(cp, p2t, precision=jax.lax.Precision.HIGHEST)
        srt = lax.transpose(sorted5, (1, 0))
        x1r, y1r = sorted5[1:2, :], sorted5[2:3, :]
        x2r, y2r = sorted5[3:4, :], sorted5[4:5, :]
        x1c, y1c = srt[:, 1:2], srt[:, 2:3]
        x2c, y2c = srt[:, 3:4], srt[:, 4:5]
        area_r = jnp.maximum(x2r - x1r, 0.0) * jnp.maximum(y2r - y1r, 0.0)
        area_c = jnp.maximum(x2c - x1c, 0.0) * jnp.maximum(y2c - y1c, 0.0)
        ix1 = jnp.maximum(x1c, x1r)
        iy1 = jnp.maximum(y1c, y1r)
        ix2 = jnp.minimum(x2c, x2r)
        iy2 = jnp.minimum(y2c, y2r)
        inter = jnp.maximum(ix2 - ix1, 0.0) * jnp.maximum(iy2 - iy1, 0.0)
        union = area_c + area_r - inter
        iou = jnp.where(union > 0.0, inter / union, 0.0)
        sup_ref[...] = ((iou > _IOU_T) & (col_i > row_i)).astype(jnp.float32)

        keep0 = (lane_i < n_sel).astype(jnp.float32)

        def blk(b, keep):
            for i in range(32):
                gi = b * 32 + i
                oh = (lane_i == gi.astype(jnp.float32)).astype(jnp.float32)
                ki = jnp.sum(keep * oh)
                rowv = sup_ref[pl.ds(gi, 1), :]
                keep = keep * (1.0 - rowv * ki)
            return keep

        keep = lax.fori_loop(0, _K // 32, blk, keep0)

        cnt = keep
        for k in (1, 2, 4, 8, 16, 32, 64, 128, 256, 512):
            cnt = cnt + jnp.concatenate(
                [jnp.zeros((1, k), jnp.float32), cnt[:, : _K - k]], axis=1)
        selv = keep * (cnt <= float(_NMSMAX)).astype(jnp.float32)

        ss = jnp.where((lane_i < n_sel) & (selv > 0.0), sorted5[0:1, :], -jnp.inf)
        msc_ref[pl.ds(c, 1), :] = ss
        sb_ref[pl.ds(c, 1)] = jnp.reshape(sorted5[1:5, :], (1, 4, _K))
        return carry

    lax.fori_loop(0, _NC, class_body, 0)

    fi = (lax.broadcasted_iota(jnp.int32, (_NC, _K), 0).astype(jnp.float32)
          * float(_K)
          + lax.broadcasted_iota(jnp.int32, (_NC, _K), 1).astype(jnp.float32))
    lane3 = lax.broadcasted_iota(jnp.int32, (1, 1, _K), 2).astype(jnp.float32)

    def ext(k, carry):
        msc = msc_ref[...]
        m = jnp.max(msc)
        eqm = msc == m
        fidx = jnp.min(jnp.where(eqm, fi, 1e9))
        cls_f = jnp.floor(fidx * (1.0 / float(_K)))
        rank_f = fidx - cls_f * float(_K)
        valid = m > _NEG
        sbc = sb_ref[pl.ds(cls_f.astype(jnp.int32), 1)]
        ohr = (lane3 == rank_f).astype(jnp.float32)
        box = jnp.sum(sbc * ohr, axis=2)
        box = jnp.where(valid, box, 0.0)
        clso = jnp.where(valid, cls_f, 0.0) + 1.0
        sco = jnp.where(valid, m, 0.0)
        row6 = jnp.concatenate(
            [jnp.reshape(clso, (1, 1)), jnp.reshape(sco, (1, 1)), box], axis=1)
        out_ref[0, pl.ds(k, 1), :] = row6
        msc_ref[...] = jnp.where(fi == fidx, -jnp.inf, msc)
        return carry

    lax.fori_loop(0, _TOPK, ext, 0)


def kernel(y_pred):
    ypt = jnp.transpose(y_pred, (0, 2, 1))
    ypt = jnp.pad(ypt, ((0, 0), (0, 0), (0, _NAP - _NA)))
    ypt4 = ypt.reshape(_NI, 33, _ROWS, 128)

    pm, sth, bxs = pl.pallas_call(
        _select_kernel,
        grid=(_NI, _NC),
        in_specs=[pl.BlockSpec((1, 33, _ROWS, 128), lambda i, c: (i, 0, 0, 0))],
        out_specs=[
            pl.BlockSpec((1, 1, _ROWS, 128), lambda i, c: (i, c, 0, 0)),
            pl.BlockSpec((1, 1, _ROWS, 128), lambda i, c: (i, c, 0, 0)),
            pl.BlockSpec((1, 4, _ROWS, 128), lambda i, c: (i, 0, 0, 0)),
        ],
        out_shape=[
            jax.ShapeDtypeStruct((_NI, _NC, _ROWS, 128), jnp.int32),
            jax.ShapeDtypeStruct((_NI, _NC, _ROWS, 128), jnp.float32),
            jax.ShapeDtypeStruct((_NI, 4, _ROWS, 128), jnp.float32),
        ],
    )(ypt4)

    comp = _sc_compact_call(
        pm.reshape(_NI * _NC * _NAP),
        sth.reshape(_NI * _NC * _NAP),
        bxs.reshape(_NI * 4 * _NAP),
    ).reshape(_NI, _NC, 5, _K)

    out = pl.pallas_call(
        _nms_kernel,
        grid=(_NI,),
        in_specs=[pl.BlockSpec((1, _NC, 5, _K), lambda i: (i, 0, 0, 0))],
        out_specs=pl.BlockSpec((1, _TOPK, 6), lambda i: (i, 0, 0)),
        out_shape=jax.ShapeDtypeStruct((_NI, _TOPK, 6), jnp.float32),
        scratch_shapes=[
            pltpu.VMEM((_K, _K), jnp.float32),
            pltpu.VMEM((_NC, _K), jnp.float32),
            pltpu.VMEM((_NC, 4, _K), jnp.float32),
        ],
    )(comp)
    return out
